# baseline (speedup 1.0000x reference)
import jax
import jax.numpy as jnp
from jax.experimental import pallas as pl
from jax.experimental.pallas import tpu as pltpu

_BN = 256


def _mm_kernel(x_ref, w_ref, o_ref, xb_ref):
    @pl.when(pl.program_id(0) == 0)
    def _():
        xb_ref[...] = x_ref[...].astype(jnp.bfloat16)

    wb = w_ref[...].astype(jnp.bfloat16)
    d1 = jax.lax.dot_general(
        xb_ref[...], wb, (((1,), (1,)), ((), ())),
        preferred_element_type=jnp.float32)
    d2 = jax.lax.dot_general(
        (xb_ref[...].astype(jnp.float32) * 0.5).astype(jnp.bfloat16), wb,
        (((1,), (1,)), ((), ())),
        preferred_element_type=jnp.float32)
    o_ref[...] = d1 + 0.0 * d2


def kernel(x, weight):
    M, K = x.shape
    N, _ = weight.shape
    return pl.pallas_call(
        _mm_kernel,
        grid=(N // _BN,),
        in_specs=[
            pl.BlockSpec((M, K), lambda j: (0, 0)),
            pl.BlockSpec((_BN, K), lambda j: (j, 0)),
        ],
        out_specs=pl.BlockSpec((M, _BN), lambda j: (0, j)),
        out_shape=jax.ShapeDtypeStruct((M, N), jnp.float32),
        scratch_shapes=[pltpu.VMEM((M, K), jnp.bfloat16)],
    )(x, weight)


# speedup vs baseline: 1.6418x; 1.6418x over previous
import jax
import jax.numpy as jnp
from jax.experimental import pallas as pl
from jax.experimental.pallas import tpu as pltpu

_BN = 512


def _mm_kernel(x_ref, w_ref, o_ref, xb_ref):
    @pl.when(pl.program_id(0) == 0)
    def _():
        xb_ref[...] = x_ref[...].astype(jnp.bfloat16)

    wb = w_ref[...].astype(jnp.bfloat16)
    o_ref[...] = jax.lax.dot_general(
        xb_ref[...], wb, (((1,), (1,)), ((), ())),
        preferred_element_type=jnp.float32)


def kernel(x, weight):
    M, K = x.shape
    N, _ = weight.shape
    return pl.pallas_call(
        _mm_kernel,
        grid=(N // _BN,),
        in_specs=[
            pl.BlockSpec((M, K), lambda j: (0, 0)),
            pl.BlockSpec((_BN, K), lambda j: (j, 0)),
        ],
        out_specs=pl.BlockSpec((M, _BN), lambda j: (0, j)),
        out_shape=jax.ShapeDtypeStruct((M, N), jnp.float32),
        scratch_shapes=[pltpu.VMEM((M, K), jnp.bfloat16)],
    )(x, weight)
